# baseline (speedup 1.0000x reference)
import functools

import jax
import jax.numpy as jnp
from jax import lax
from jax.experimental import pallas as pl
from jax.experimental.pallas import tpu as pltpu
from jax.experimental.pallas import tpu_sc as plsc

_BETA = 0.25

_BM = 1024
_BK = 8192

_NC = 2
_NS = 16
_NW = _NC * _NS
_CHUNK = 128


def _csq_body(c_ref, o_ref):
    c = c_ref[...]
    o_ref[...] = jnp.sum(c * c, axis=1, keepdims=True)


_STRIP = 16


_SUBK = 2048


def _argmin_body(nk, bk, bm, z_ref, c_ref, csq_ref, idx_ref, lsum_ref,
                 min_s, arg_s, zsq_s):
    k = pl.program_id(1)
    z = z_ref[...]

    @pl.when(k == 0)
    def _():
        zsq_s[...] = jnp.sum(z * z, axis=1, keepdims=True)

    zsq = zsq_s[...]
    sub = min(_SUBK, bk)
    cols = lax.broadcasted_iota(jnp.int32, (bm, sub), 1).astype(jnp.float32)

    run_min = run_arg = None
    for g in range(bk // sub):
        c_g = c_ref[pl.ds(g * sub, sub), :]
        m = lax.dot_general(z, c_g, (((1,), (1,)), ((), ())),
                            preferred_element_type=jnp.float32)
        d = (zsq + csq_ref[:, pl.ds(g * sub, sub)]) - 2.0 * m
        lmin = jnp.min(d, axis=1, keepdims=True)
        lidx = jnp.min(jnp.where(d == lmin, cols, float(sub)), axis=1,
                       keepdims=True)
        larg = lidx.astype(jnp.int32) + (k * bk + g * sub)
        if run_min is None:
            run_min, run_arg = lmin, larg
        else:
            t = lmin < run_min
            run_min = jnp.where(t, lmin, run_min)
            run_arg = jnp.where(t, larg, run_arg)

    prev_min = min_s[...]
    prev_arg = arg_s[...]
    take = jnp.logical_or(run_min < prev_min, k == 0)
    min_s[...] = jnp.where(take, run_min, prev_min)
    arg_s[...] = jnp.where(take, run_arg, prev_arg)

    @pl.when(k == nk - 1)
    def _():
        idx_ref[...] = arg_s[...][None]
        lsum_ref[...] = jnp.sum(min_s[...])[None, None, None]


def _distance_argmin(z, codebook):
    b, d_model = z.shape
    k_size, _ = codebook.shape
    bm, bk = min(_BM, b), min(_BK, k_size)
    nb, nk = b // bm, k_size // bk

    csq_col = pl.pallas_call(
        _csq_body,
        grid=(nk,),
        in_specs=[pl.BlockSpec((bk, d_model), lambda j: (j, 0))],
        out_specs=pl.BlockSpec((bk, 1), lambda j: (j, 0)),
        out_shape=jax.ShapeDtypeStruct((k_size, 1), jnp.float32),
    )(codebook)
    csq_row = csq_col.reshape(1, k_size)

    idx3, lsum = pl.pallas_call(
        functools.partial(_argmin_body, nk, bk, bm),
        grid=(nb, nk),
        in_specs=[
            pl.BlockSpec((bm, d_model), lambda i, j: (i, 0)),
            pl.BlockSpec((bk, d_model), lambda i, j: (j, 0)),
            pl.BlockSpec((1, bk), lambda i, j: (0, j)),
        ],
        out_specs=[
            pl.BlockSpec((1, bm, 1), lambda i, j: (i, 0, 0)),
            pl.BlockSpec((1, 1, 1), lambda i, j: (i, 0, 0)),
        ],
        out_shape=[
            jax.ShapeDtypeStruct((nb, bm, 1), jnp.int32),
            jax.ShapeDtypeStruct((nb, 1, 1), jnp.float32),
        ],
        scratch_shapes=[
            pltpu.VMEM((bm, 1), jnp.float32),
            pltpu.VMEM((bm, 1), jnp.int32),
            pltpu.VMEM((bm, 1), jnp.float32),
        ],
        compiler_params=pltpu.CompilerParams(
            dimension_semantics=("parallel", "arbitrary"),
        ),
    )(z, codebook, csq_row)
    return idx3.reshape(b), lsum.reshape(nb)


def _gather_body(n_chunk, b_per_w, cb_hbm, idx_hbm, out_hbm,
                 idx_v, rows_v, sem):
    wid = lax.axis_index("s") * _NC + lax.axis_index("c")
    base = wid * b_per_w
    for ci in range(n_chunk):
        cbase = base + ci * _CHUNK
        pltpu.sync_copy(idx_hbm.at[pl.ds(cbase, _CHUNK)], idx_v)
        pltpu.async_copy(cb_hbm.at[idx_v], rows_v, sem).wait()
        pltpu.sync_copy(rows_v, out_hbm.at[pl.ds(cbase, _CHUNK)])


def _gather_rows(codebook, idx):
    b = idx.shape[0]
    d_model = codebook.shape[1]
    b_per_w = b // _NW
    n_chunk = b_per_w // _CHUNK

    mesh = plsc.VectorSubcoreMesh(core_axis_name="c", subcore_axis_name="s")
    fn = functools.partial(
        pl.kernel,
        mesh=mesh,
        out_type=jax.ShapeDtypeStruct((b, d_model), jnp.float32),
        scratch_types=[
            pltpu.VMEM((_CHUNK,), jnp.int32),
            pltpu.VMEM((_CHUNK, d_model), jnp.float32),
            pltpu.SemaphoreType.DMA,
        ],
    )(functools.partial(_gather_body, n_chunk, b_per_w))
    return fn(codebook, idx)


def kernel(z, codebook):
    b, d_model = z.shape
    idx, lsum = _distance_argmin(z, codebook)
    z_q = _gather_rows(codebook, idx)
    loss = jnp.sum(lsum) / (b * d_model)
    vq_loss = loss + _BETA * loss
    return (z_q, idx, vq_loss)


# speedup vs baseline: 1.0489x; 1.0489x over previous
import functools

import jax
import jax.numpy as jnp
from jax import lax
from jax.experimental import pallas as pl
from jax.experimental.pallas import tpu as pltpu
from jax.experimental.pallas import tpu_sc as plsc

_BETA = 0.25

_BM = 512
_BK = 8192

_NC = 2
_NS = 16
_NW = _NC * _NS
_CHUNK = 128


def _csq_body(c_ref, o_ref):
    c = c_ref[...]
    o_ref[...] = jnp.sum(c * c, axis=1, keepdims=True)


_STRIP = 16


_SUBK = 8192


def _argmin_body(nk, bk, bm, z_ref, c_ref, csq_ref, idx_ref, lsum_ref,
                 min_s, arg_s, zsq_s):
    k = pl.program_id(1)
    z = z_ref[...]

    @pl.when(k == 0)
    def _():
        zsq_s[...] = jnp.sum(z * z, axis=1, keepdims=True)

    zsq = zsq_s[...]
    sub = min(_SUBK, bk)
    cols = lax.broadcasted_iota(jnp.int32, (bm, sub), 1).astype(jnp.float32)

    run_min = run_arg = None
    for g in range(bk // sub):
        c_g = c_ref[pl.ds(g * sub, sub), :]
        m = lax.dot_general(z, c_g, (((1,), (1,)), ((), ())),
                            preferred_element_type=jnp.float32)
        d = (zsq + csq_ref[:, pl.ds(g * sub, sub)]) - 2.0 * m
        lmin = jnp.min(d, axis=1, keepdims=True)
        lidx = jnp.min(jnp.where(d == lmin, cols, float(sub)), axis=1,
                       keepdims=True)
        larg = lidx.astype(jnp.int32) + (k * bk + g * sub)
        if run_min is None:
            run_min, run_arg = lmin, larg
        else:
            t = lmin < run_min
            run_min = jnp.where(t, lmin, run_min)
            run_arg = jnp.where(t, larg, run_arg)

    prev_min = min_s[...]
    prev_arg = arg_s[...]
    take = jnp.logical_or(run_min < prev_min, k == 0)
    min_s[...] = jnp.where(take, run_min, prev_min)
    arg_s[...] = jnp.where(take, run_arg, prev_arg)

    @pl.when(k == nk - 1)
    def _():
        idx_ref[...] = arg_s[...][None]
        lsum_ref[...] = jnp.sum(min_s[...])[None, None, None]


def _distance_argmin(z, codebook):
    b, d_model = z.shape
    k_size, _ = codebook.shape
    bm, bk = min(_BM, b), min(_BK, k_size)
    nb, nk = b // bm, k_size // bk

    csq_col = pl.pallas_call(
        _csq_body,
        grid=(nk,),
        in_specs=[pl.BlockSpec((bk, d_model), lambda j: (j, 0))],
        out_specs=pl.BlockSpec((bk, 1), lambda j: (j, 0)),
        out_shape=jax.ShapeDtypeStruct((k_size, 1), jnp.float32),
    )(codebook)
    csq_row = csq_col.reshape(1, k_size)

    idx3, lsum = pl.pallas_call(
        functools.partial(_argmin_body, nk, bk, bm),
        grid=(nb, nk),
        in_specs=[
            pl.BlockSpec((bm, d_model), lambda i, j: (i, 0)),
            pl.BlockSpec((bk, d_model), lambda i, j: (j, 0)),
            pl.BlockSpec((1, bk), lambda i, j: (0, j)),
        ],
        out_specs=[
            pl.BlockSpec((1, bm, 1), lambda i, j: (i, 0, 0)),
            pl.BlockSpec((1, 1, 1), lambda i, j: (i, 0, 0)),
        ],
        out_shape=[
            jax.ShapeDtypeStruct((nb, bm, 1), jnp.int32),
            jax.ShapeDtypeStruct((nb, 1, 1), jnp.float32),
        ],
        scratch_shapes=[
            pltpu.VMEM((bm, 1), jnp.float32),
            pltpu.VMEM((bm, 1), jnp.int32),
            pltpu.VMEM((bm, 1), jnp.float32),
        ],
        compiler_params=pltpu.CompilerParams(
            dimension_semantics=("parallel", "arbitrary"),
        ),
    )(z, codebook, csq_row)
    return idx3.reshape(b), lsum.reshape(nb)


def _gather_body(n_chunk, b_per_w, cb_hbm, idx_hbm, out_hbm,
                 idx_v, rows_v, sem):
    wid = lax.axis_index("s") * _NC + lax.axis_index("c")
    base = wid * b_per_w
    for ci in range(n_chunk):
        cbase = base + ci * _CHUNK
        pltpu.sync_copy(idx_hbm.at[pl.ds(cbase, _CHUNK)], idx_v)
        pltpu.async_copy(cb_hbm.at[idx_v], rows_v, sem).wait()
        pltpu.sync_copy(rows_v, out_hbm.at[pl.ds(cbase, _CHUNK)])


def _gather_rows(codebook, idx):
    b = idx.shape[0]
    d_model = codebook.shape[1]
    b_per_w = b // _NW
    n_chunk = b_per_w // _CHUNK

    mesh = plsc.VectorSubcoreMesh(core_axis_name="c", subcore_axis_name="s")
    fn = functools.partial(
        pl.kernel,
        mesh=mesh,
        out_type=jax.ShapeDtypeStruct((b, d_model), jnp.float32),
        scratch_types=[
            pltpu.VMEM((_CHUNK,), jnp.int32),
            pltpu.VMEM((_CHUNK, d_model), jnp.float32),
            pltpu.SemaphoreType.DMA,
        ],
    )(functools.partial(_gather_body, n_chunk, b_per_w))
    return fn(codebook, idx)


def kernel(z, codebook):
    b, d_model = z.shape
    idx, lsum = _distance_argmin(z, codebook)
    z_q = _gather_rows(codebook, idx)
    loss = jnp.sum(lsum) / (b * d_model)
    vq_loss = loss + _BETA * loss
    return (z_q, idx, vq_loss)


# speedup vs baseline: 1.0571x; 1.0079x over previous
import functools

import jax
import jax.numpy as jnp
from jax import lax
from jax.experimental import pallas as pl
from jax.experimental.pallas import tpu as pltpu
from jax.experimental.pallas import tpu_sc as plsc

_BETA = 0.25

_BM = 1024
_BK = 8192

_NC = 2
_NS = 16
_NW = _NC * _NS
_CHUNK = 128


def _csq_body(c_ref, o_ref):
    c = c_ref[...]
    o_ref[...] = jnp.sum(c * c, axis=1, keepdims=True)


_STRIP = 16


_SUBK = 4096


def _argmin_body(nk, bk, bm, z_ref, c_ref, csq_ref, idx_ref, lsum_ref,
                 min_s, arg_s, zsq_s):
    k = pl.program_id(1)
    z = z_ref[...]

    @pl.when(k == 0)
    def _():
        zsq_s[...] = jnp.sum(z * z, axis=1, keepdims=True)

    zsq = zsq_s[...]
    sub = min(_SUBK, bk)
    cols = lax.broadcasted_iota(jnp.int32, (bm, sub), 1).astype(jnp.float32)

    run_min = run_arg = None
    for g in range(bk // sub):
        c_g = c_ref[pl.ds(g * sub, sub), :]
        m = lax.dot_general(z, c_g, (((1,), (1,)), ((), ())),
                            preferred_element_type=jnp.float32)
        d = (zsq + csq_ref[:, pl.ds(g * sub, sub)]) - 2.0 * m
        lmin = jnp.min(d, axis=1, keepdims=True)
        lidx = jnp.min(jnp.where(d == lmin, cols, float(sub)), axis=1,
                       keepdims=True)
        larg = lidx.astype(jnp.int32) + (k * bk + g * sub)
        if run_min is None:
            run_min, run_arg = lmin, larg
        else:
            t = lmin < run_min
            run_min = jnp.where(t, lmin, run_min)
            run_arg = jnp.where(t, larg, run_arg)

    prev_min = min_s[...]
    prev_arg = arg_s[...]
    take = jnp.logical_or(run_min < prev_min, k == 0)
    min_s[...] = jnp.where(take, run_min, prev_min)
    arg_s[...] = jnp.where(take, run_arg, prev_arg)

    @pl.when(k == nk - 1)
    def _():
        idx_ref[...] = arg_s[...][None]
        lsum_ref[...] = jnp.sum(min_s[...])[None, None, None]


def _distance_argmin(z, codebook):
    b, d_model = z.shape
    k_size, _ = codebook.shape
    bm, bk = min(_BM, b), min(_BK, k_size)
    nb, nk = b // bm, k_size // bk

    csq_col = pl.pallas_call(
        _csq_body,
        grid=(nk,),
        in_specs=[pl.BlockSpec((bk, d_model), lambda j: (j, 0))],
        out_specs=pl.BlockSpec((bk, 1), lambda j: (j, 0)),
        out_shape=jax.ShapeDtypeStruct((k_size, 1), jnp.float32),
    )(codebook)
    csq_row = csq_col.reshape(1, k_size)

    idx3, lsum = pl.pallas_call(
        functools.partial(_argmin_body, nk, bk, bm),
        grid=(nb, nk),
        in_specs=[
            pl.BlockSpec((bm, d_model), lambda i, j: (i, 0)),
            pl.BlockSpec((bk, d_model), lambda i, j: (j, 0)),
            pl.BlockSpec((1, bk), lambda i, j: (0, j)),
        ],
        out_specs=[
            pl.BlockSpec((1, bm, 1), lambda i, j: (i, 0, 0)),
            pl.BlockSpec((1, 1, 1), lambda i, j: (i, 0, 0)),
        ],
        out_shape=[
            jax.ShapeDtypeStruct((nb, bm, 1), jnp.int32),
            jax.ShapeDtypeStruct((nb, 1, 1), jnp.float32),
        ],
        scratch_shapes=[
            pltpu.VMEM((bm, 1), jnp.float32),
            pltpu.VMEM((bm, 1), jnp.int32),
            pltpu.VMEM((bm, 1), jnp.float32),
        ],
        compiler_params=pltpu.CompilerParams(
            dimension_semantics=("parallel", "arbitrary"),
        ),
    )(z, codebook, csq_row)
    return idx3.reshape(b), lsum.reshape(nb)


def _gather_body(n_chunk, b_per_w, cb_hbm, idx_hbm, out_hbm,
                 idx_v, rows_v, sem):
    wid = lax.axis_index("s") * _NC + lax.axis_index("c")
    base = wid * b_per_w
    for ci in range(n_chunk):
        cbase = base + ci * _CHUNK
        pltpu.sync_copy(idx_hbm.at[pl.ds(cbase, _CHUNK)], idx_v)
        pltpu.async_copy(cb_hbm.at[idx_v], rows_v, sem).wait()
        pltpu.sync_copy(rows_v, out_hbm.at[pl.ds(cbase, _CHUNK)])


def _gather_rows(codebook, idx):
    b = idx.shape[0]
    d_model = codebook.shape[1]
    b_per_w = b // _NW
    n_chunk = b_per_w // _CHUNK

    mesh = plsc.VectorSubcoreMesh(core_axis_name="c", subcore_axis_name="s")
    fn = functools.partial(
        pl.kernel,
        mesh=mesh,
        out_type=jax.ShapeDtypeStruct((b, d_model), jnp.float32),
        scratch_types=[
            pltpu.VMEM((_CHUNK,), jnp.int32),
            pltpu.VMEM((_CHUNK, d_model), jnp.float32),
            pltpu.SemaphoreType.DMA,
        ],
    )(functools.partial(_gather_body, n_chunk, b_per_w))
    return fn(codebook, idx)


def kernel(z, codebook):
    b, d_model = z.shape
    idx, lsum = _distance_argmin(z, codebook)
    z_q = _gather_rows(codebook, idx)
    loss = jnp.sum(lsum) / (b * d_model)
    vq_loss = loss + _BETA * loss
    return (z_q, idx, vq_loss)


# speedup vs baseline: 1.0875x; 1.0287x over previous
import functools

import jax
import jax.numpy as jnp
from jax import lax
from jax.experimental import pallas as pl
from jax.experimental.pallas import tpu as pltpu
from jax.experimental.pallas import tpu_sc as plsc

_BETA = 0.25

_BM = 1024
_BK = 8192

_NC = 2
_NS = 16
_NW = _NC * _NS
_CHUNK = 128


def _csq_body(c_ref, o_ref):
    c = c_ref[...]
    o_ref[...] = jnp.sum(c * c, axis=1, keepdims=True)


_SUBK = 8192


def _argmin_body(nk, bk, bm, z_ref, c_ref, csq_ref, idx_ref, lsum_ref,
                 min_s, arg_s, zsq_s):
    k = pl.program_id(1)
    z = z_ref[...]

    @pl.when(k == 0)
    def _():
        zsq_s[...] = jnp.sum(z * z, axis=1, keepdims=True)

    zsq = zsq_s[...]
    sub = min(_SUBK, bk)
    cols = lax.broadcasted_iota(jnp.int32, (bm, sub), 1).astype(jnp.float32)

    run_min = run_arg = None
    for g in range(bk // sub):
        c_g = c_ref[pl.ds(g * sub, sub), :]
        m = lax.dot_general(z, c_g, (((1,), (1,)), ((), ())),
                            preferred_element_type=jnp.float32)
        d = (zsq + csq_ref[:, pl.ds(g * sub, sub)]) - 2.0 * m
        lmin = jnp.min(d, axis=1, keepdims=True)
        lidx = jnp.min(jnp.where(d == lmin, cols, float(sub)), axis=1,
                       keepdims=True)
        larg = lidx.astype(jnp.int32) + (k * bk + g * sub)
        if run_min is None:
            run_min, run_arg = lmin, larg
        else:
            t = lmin < run_min
            run_min = jnp.where(t, lmin, run_min)
            run_arg = jnp.where(t, larg, run_arg)

    prev_min = min_s[...]
    prev_arg = arg_s[...]
    take = jnp.logical_or(run_min < prev_min, k == 0)
    min_s[...] = jnp.where(take, run_min, prev_min)
    arg_s[...] = jnp.where(take, run_arg, prev_arg)

    @pl.when(k == nk - 1)
    def _():
        idx_ref[...] = arg_s[...][None]
        lsum_ref[...] = jnp.sum(min_s[...])[None, None, None]


def _distance_argmin(z, codebook):
    b, d_model = z.shape
    k_size, _ = codebook.shape
    bm, bk = min(_BM, b), min(_BK, k_size)
    nb, nk = b // bm, k_size // bk

    csq_col = pl.pallas_call(
        _csq_body,
        grid=(nk,),
        in_specs=[pl.BlockSpec((bk, d_model), lambda j: (j, 0))],
        out_specs=pl.BlockSpec((bk, 1), lambda j: (j, 0)),
        out_shape=jax.ShapeDtypeStruct((k_size, 1), jnp.float32),
    )(codebook)
    csq_row = csq_col.reshape(1, k_size)

    idx3, lsum = pl.pallas_call(
        functools.partial(_argmin_body, nk, bk, bm),
        grid=(nb, nk),
        in_specs=[
            pl.BlockSpec((bm, d_model), lambda i, j: (i, 0)),
            pl.BlockSpec((bk, d_model), lambda i, j: (j, 0)),
            pl.BlockSpec((1, bk), lambda i, j: (0, j)),
        ],
        out_specs=[
            pl.BlockSpec((1, bm, 1), lambda i, j: (i, 0, 0)),
            pl.BlockSpec((1, 1, 1), lambda i, j: (i, 0, 0)),
        ],
        out_shape=[
            jax.ShapeDtypeStruct((nb, bm, 1), jnp.int32),
            jax.ShapeDtypeStruct((nb, 1, 1), jnp.float32),
        ],
        scratch_shapes=[
            pltpu.VMEM((bm, 1), jnp.float32),
            pltpu.VMEM((bm, 1), jnp.int32),
            pltpu.VMEM((bm, 1), jnp.float32),
        ],
        compiler_params=pltpu.CompilerParams(
            dimension_semantics=("parallel", "arbitrary"),
        ),
    )(z, codebook, csq_row)
    return idx3.reshape(b), lsum.reshape(nb)


def _gather_body(n_chunk, b_per_w, cb_hbm, idx_hbm, out_hbm,
                 idx_v, rows_v, sem):
    wid = lax.axis_index("s") * _NC + lax.axis_index("c")
    base = wid * b_per_w
    for ci in range(n_chunk):
        cbase = base + ci * _CHUNK
        pltpu.sync_copy(idx_hbm.at[pl.ds(cbase, _CHUNK)], idx_v)
        pltpu.async_copy(cb_hbm.at[idx_v], rows_v, sem).wait()
        pltpu.sync_copy(rows_v, out_hbm.at[pl.ds(cbase, _CHUNK)])


def _gather_rows(codebook, idx):
    b = idx.shape[0]
    d_model = codebook.shape[1]
    b_per_w = b // _NW
    n_chunk = b_per_w // _CHUNK

    mesh = plsc.VectorSubcoreMesh(core_axis_name="c", subcore_axis_name="s")
    fn = functools.partial(
        pl.kernel,
        mesh=mesh,
        out_type=jax.ShapeDtypeStruct((b, d_model), jnp.float32),
        scratch_types=[
            pltpu.VMEM((_CHUNK,), jnp.int32),
            pltpu.VMEM((_CHUNK, d_model), jnp.float32),
            pltpu.SemaphoreType.DMA,
        ],
    )(functools.partial(_gather_body, n_chunk, b_per_w))
    return fn(codebook, idx)


def kernel(z, codebook):
    b, d_model = z.shape
    idx, lsum = _distance_argmin(z, codebook)
    z_q = _gather_rows(codebook, idx)
    loss = jnp.sum(lsum) / (b * d_model)
    vq_loss = loss + _BETA * loss
    return (z_q, idx, vq_loss)
